# baseline (speedup 1.0000x reference)
import functools

import jax
import jax.numpy as jnp
from jax.experimental import pallas as pl
from jax.experimental.pallas import tpu as pltpu

DIM = 256
CODEBOOK_SIZE = 1024
NUM_CODEBOOKS = 8
N_TOKENS = 16384
COMMIT_COST = 0.25

TILE = 2048


def _rvq_kernel(z_ref, cb_ref, cb_hi_ref, cb_m_ref, cb_l_ref,
                zq_ref, codes_ref, sse_ref):
    z = z_ref[...]
    r = z
    qs = jnp.zeros_like(z)
    iota = jax.lax.broadcasted_iota(jnp.int32, (TILE, CODEBOOK_SIZE), 1)
    for i in range(NUM_CODEBOOKS):
        cb = cb_ref[i]
        x_norm = jnp.sum(r * r, axis=1, keepdims=True)
        c_norm = jnp.sum(cb * cb, axis=1)[None, :]
        dots = jax.lax.dot_general(
            r.astype(jnp.bfloat16), cb.astype(jnp.bfloat16),
            (((1,), (1,)), ((), ())),
            preferred_element_type=jnp.float32)
        d = x_norm - 2.0 * dots + c_norm
        dmin = jnp.min(d, axis=1, keepdims=True)
        idx = jnp.min(jnp.where(d == dmin, iota, CODEBOOK_SIZE), axis=1)
        codes_ref[:, i] = idx
        onehot = (iota == idx[:, None]).astype(jnp.bfloat16)
        q_hi = jax.lax.dot_general(
            onehot, cb_hi_ref[i], (((1,), (0,)), ((), ())),
            preferred_element_type=jnp.float32)
        q_m = jax.lax.dot_general(
            onehot, cb_m_ref[i], (((1,), (0,)), ((), ())),
            preferred_element_type=jnp.float32)
        q_l = jax.lax.dot_general(
            onehot, cb_l_ref[i], (((1,), (0,)), ((), ())),
            preferred_element_type=jnp.float32)
        bits = (jax.lax.bitcast_convert_type(q_hi, jnp.int32)
                | (q_m.astype(jnp.int32) << 8)
                | q_l.astype(jnp.int32))
        q = jax.lax.bitcast_convert_type(bits, jnp.float32)
        qs = qs + q
        r = r - q
    zq_ref[...] = z + (qs - z)
    diff = qs - z
    sse_ref[...] = jnp.full((1, 1, 128), jnp.sum(diff * diff),
                            dtype=jnp.float32)


@jax.jit
def kernel(z_e, codebooks):
    n_tiles = N_TOKENS // TILE
    cb_bits = jax.lax.bitcast_convert_type(codebooks, jnp.int32)
    cb_hi = jax.lax.bitcast_convert_type(
        (cb_bits >> 16).astype(jnp.uint16), jnp.bfloat16)
    cb_m = ((cb_bits >> 8) & 0xFF).astype(jnp.bfloat16)
    cb_l = (cb_bits & 0xFF).astype(jnp.bfloat16)

    cb_spec = pl.BlockSpec((NUM_CODEBOOKS, CODEBOOK_SIZE, DIM),
                           lambda i: (0, 0, 0))
    zq, codes, sse = pl.pallas_call(
        _rvq_kernel,
        grid=(n_tiles,),
        in_specs=[
            pl.BlockSpec((TILE, DIM), lambda i: (i, 0)),
            cb_spec, cb_spec, cb_spec, cb_spec,
        ],
        out_specs=[
            pl.BlockSpec((TILE, DIM), lambda i: (i, 0)),
            pl.BlockSpec((TILE, NUM_CODEBOOKS), lambda i: (i, 0)),
            pl.BlockSpec((1, 1, 128), lambda i: (i, 0, 0)),
        ],
        out_shape=[
            jax.ShapeDtypeStruct((N_TOKENS, DIM), jnp.float32),
            jax.ShapeDtypeStruct((N_TOKENS, NUM_CODEBOOKS), jnp.int32),
            jax.ShapeDtypeStruct((n_tiles, 1, 128), jnp.float32),
        ],
        compiler_params=pltpu.CompilerParams(
            dimension_semantics=("parallel",)),
    )(z_e, codebooks, cb_hi, cb_m, cb_l)
    loss = (1.0 + COMMIT_COST) * jnp.sum(sse[:, 0, 0]) / (N_TOKENS * DIM)
    return (zq, codes, loss)


# speedup vs baseline: 1.0136x; 1.0136x over previous
import functools

import jax
import jax.numpy as jnp
from jax.experimental import pallas as pl
from jax.experimental.pallas import tpu as pltpu

DIM = 256
CODEBOOK_SIZE = 1024
NUM_CODEBOOKS = 8
N_TOKENS = 16384
COMMIT_COST = 0.25

TILE = 512


def _rvq_kernel(z_ref, cb_ref, cb_hi_ref, cb_m_ref, cb_l_ref,
                zq_ref, codes_ref, sse_ref):
    z = z_ref[...]
    r = z
    qs = jnp.zeros_like(z)
    iota = jax.lax.broadcasted_iota(jnp.int32, (TILE, CODEBOOK_SIZE), 1)
    for i in range(NUM_CODEBOOKS):
        cb = cb_ref[i]
        x_norm = jnp.sum(r * r, axis=1, keepdims=True)
        c_norm = jnp.sum(cb * cb, axis=1)[None, :]
        dots = jax.lax.dot_general(
            r.astype(jnp.bfloat16), cb.astype(jnp.bfloat16),
            (((1,), (1,)), ((), ())),
            preferred_element_type=jnp.float32)
        d = x_norm - 2.0 * dots + c_norm
        dmin = jnp.min(d, axis=1, keepdims=True)
        idx = jnp.min(jnp.where(d == dmin, iota, CODEBOOK_SIZE), axis=1)
        codes_ref[:, i] = idx
        onehot = (iota == idx[:, None]).astype(jnp.bfloat16)
        q_hi = jax.lax.dot_general(
            onehot, cb_hi_ref[i], (((1,), (0,)), ((), ())),
            preferred_element_type=jnp.float32)
        q_m = jax.lax.dot_general(
            onehot, cb_m_ref[i], (((1,), (0,)), ((), ())),
            preferred_element_type=jnp.float32)
        q_l = jax.lax.dot_general(
            onehot, cb_l_ref[i], (((1,), (0,)), ((), ())),
            preferred_element_type=jnp.float32)
        bits = (jax.lax.bitcast_convert_type(q_hi, jnp.int32)
                | (q_m.astype(jnp.int32) << 8)
                | q_l.astype(jnp.int32))
        q = jax.lax.bitcast_convert_type(bits, jnp.float32)
        qs = qs + q
        r = r - q
    zq_ref[...] = z + (qs - z)
    diff = qs - z
    sse_ref[...] = jnp.full((1, 1, 128), jnp.sum(diff * diff),
                            dtype=jnp.float32)


@jax.jit
def kernel(z_e, codebooks):
    n_tiles = N_TOKENS // TILE
    cb_bits = jax.lax.bitcast_convert_type(codebooks, jnp.int32)
    cb_hi = jax.lax.bitcast_convert_type(
        (cb_bits >> 16).astype(jnp.uint16), jnp.bfloat16)
    cb_m = ((cb_bits >> 8) & 0xFF).astype(jnp.bfloat16)
    cb_l = (cb_bits & 0xFF).astype(jnp.bfloat16)

    cb_spec = pl.BlockSpec((NUM_CODEBOOKS, CODEBOOK_SIZE, DIM),
                           lambda i: (0, 0, 0))
    zq, codes, sse = pl.pallas_call(
        _rvq_kernel,
        grid=(n_tiles,),
        in_specs=[
            pl.BlockSpec((TILE, DIM), lambda i: (i, 0)),
            cb_spec, cb_spec, cb_spec, cb_spec,
        ],
        out_specs=[
            pl.BlockSpec((TILE, DIM), lambda i: (i, 0)),
            pl.BlockSpec((TILE, NUM_CODEBOOKS), lambda i: (i, 0)),
            pl.BlockSpec((1, 1, 128), lambda i: (i, 0, 0)),
        ],
        out_shape=[
            jax.ShapeDtypeStruct((N_TOKENS, DIM), jnp.float32),
            jax.ShapeDtypeStruct((N_TOKENS, NUM_CODEBOOKS), jnp.int32),
            jax.ShapeDtypeStruct((n_tiles, 1, 128), jnp.float32),
        ],
        compiler_params=pltpu.CompilerParams(
            dimension_semantics=("parallel",)),
    )(z_e, codebooks, cb_hi, cb_m, cb_l)
    loss = (1.0 + COMMIT_COST) * jnp.sum(sse[:, 0, 0]) / (N_TOKENS * DIM)
    return (zq, codes, loss)


# speedup vs baseline: 1.1393x; 1.1241x over previous
import functools

import jax
import jax.numpy as jnp
from jax.experimental import pallas as pl
from jax.experimental.pallas import tpu as pltpu

DIM = 256
CODEBOOK_SIZE = 1024
NUM_CODEBOOKS = 8
N_TOKENS = 16384
COMMIT_COST = 0.25

TILE = 1024


def _rvq_kernel(z_ref, cb_ref, cb_hi_ref, cb_m_ref, cb_l_ref,
                zq_ref, codes_ref, sse_ref):
    z = z_ref[...]
    r = z
    qs = jnp.zeros_like(z)
    iota = jax.lax.broadcasted_iota(jnp.int32, (TILE, CODEBOOK_SIZE), 1)
    for i in range(NUM_CODEBOOKS):
        cb = cb_ref[i]
        x_norm = jnp.sum(r * r, axis=1, keepdims=True)
        c_norm = jnp.sum(cb * cb, axis=1)[None, :]
        dots = jax.lax.dot_general(
            r.astype(jnp.bfloat16), cb.astype(jnp.bfloat16),
            (((1,), (1,)), ((), ())),
            preferred_element_type=jnp.float32)
        d = x_norm - 2.0 * dots + c_norm
        dmin = jnp.min(d, axis=1, keepdims=True)
        idx = jnp.min(jnp.where(d == dmin, iota, CODEBOOK_SIZE), axis=1)
        codes_ref[:, i] = idx
        onehot = (iota == idx[:, None]).astype(jnp.bfloat16)
        q_hi = jax.lax.dot_general(
            onehot, cb_hi_ref[i], (((1,), (0,)), ((), ())),
            preferred_element_type=jnp.float32)
        q_m = jax.lax.dot_general(
            onehot, cb_m_ref[i], (((1,), (0,)), ((), ())),
            preferred_element_type=jnp.float32)
        q_l = jax.lax.dot_general(
            onehot, cb_l_ref[i], (((1,), (0,)), ((), ())),
            preferred_element_type=jnp.float32)
        bits = (jax.lax.bitcast_convert_type(q_hi, jnp.int32)
                | (q_m.astype(jnp.int32) << 8)
                | q_l.astype(jnp.int32))
        q = jax.lax.bitcast_convert_type(bits, jnp.float32)
        qs = qs + q
        r = r - q
    zq_ref[...] = z + (qs - z)
    diff = qs - z
    sse_ref[...] = jnp.full((1, 1, 128), jnp.sum(diff * diff),
                            dtype=jnp.float32)


@jax.jit
def kernel(z_e, codebooks):
    n_tiles = N_TOKENS // TILE
    cb_bits = jax.lax.bitcast_convert_type(codebooks, jnp.int32)
    cb_hi = jax.lax.bitcast_convert_type(
        (cb_bits >> 16).astype(jnp.uint16), jnp.bfloat16)
    cb_m = ((cb_bits >> 8) & 0xFF).astype(jnp.bfloat16)
    cb_l = (cb_bits & 0xFF).astype(jnp.bfloat16)

    cb_spec = pl.BlockSpec((NUM_CODEBOOKS, CODEBOOK_SIZE, DIM),
                           lambda i: (0, 0, 0))
    zq, codes, sse = pl.pallas_call(
        _rvq_kernel,
        grid=(n_tiles,),
        in_specs=[
            pl.BlockSpec((TILE, DIM), lambda i: (i, 0)),
            cb_spec, cb_spec, cb_spec, cb_spec,
        ],
        out_specs=[
            pl.BlockSpec((TILE, DIM), lambda i: (i, 0)),
            pl.BlockSpec((TILE, NUM_CODEBOOKS), lambda i: (i, 0)),
            pl.BlockSpec((1, 1, 128), lambda i: (i, 0, 0)),
        ],
        out_shape=[
            jax.ShapeDtypeStruct((N_TOKENS, DIM), jnp.float32),
            jax.ShapeDtypeStruct((N_TOKENS, NUM_CODEBOOKS), jnp.int32),
            jax.ShapeDtypeStruct((n_tiles, 1, 128), jnp.float32),
        ],
        compiler_params=pltpu.CompilerParams(
            dimension_semantics=("parallel",)),
    )(z_e, codebooks, cb_hi, cb_m, cb_l)
    loss = (1.0 + COMMIT_COST) * jnp.sum(sse[:, 0, 0]) / (N_TOKENS * DIM)
    return (zq, codes, loss)


# speedup vs baseline: 1.1417x; 1.0021x over previous
import functools

import jax
import jax.numpy as jnp
from jax.experimental import pallas as pl
from jax.experimental.pallas import tpu as pltpu

DIM = 256
CODEBOOK_SIZE = 1024
NUM_CODEBOOKS = 8
N_TOKENS = 16384
COMMIT_COST = 0.25

TILE = 1024


def _rvq_kernel(z_ref, cb_ref, cb_hi_ref, cb_m_ref, cb_l_ref,
                zq_ref, codes_ref, sse_ref):
    z = z_ref[...]
    r = z
    qs = jnp.zeros_like(z)
    iota = jax.lax.broadcasted_iota(jnp.int32, (TILE, CODEBOOK_SIZE), 1)
    for i in range(NUM_CODEBOOKS):
        cb = cb_ref[i]
        x_norm = jnp.sum(r * r, axis=1, keepdims=True)
        c_norm = jnp.sum(cb * cb, axis=1)[None, :]
        dots = jax.lax.dot_general(
            r.astype(jnp.bfloat16), cb.astype(jnp.bfloat16),
            (((1,), (1,)), ((), ())),
            preferred_element_type=jnp.float32)
        d = x_norm - 2.0 * dots + c_norm
        dmin = jnp.min(d, axis=1, keepdims=True)
        idx = jnp.min(jnp.where(d == dmin, iota, CODEBOOK_SIZE), axis=1)
        codes_ref[:, i] = idx
        onehot = (iota == idx[:, None]).astype(jnp.bfloat16)
        q_hi = jax.lax.dot_general(
            onehot, cb_hi_ref[i], (((1,), (0,)), ((), ())),
            preferred_element_type=jnp.float32)
        q_m = jax.lax.dot_general(
            onehot, cb_m_ref[i], (((1,), (0,)), ((), ())),
            preferred_element_type=jnp.float32)
        q_l = jax.lax.dot_general(
            onehot, cb_l_ref[i], (((1,), (0,)), ((), ())),
            preferred_element_type=jnp.float32)
        bits = (jax.lax.bitcast_convert_type(q_hi, jnp.int32)
                | (q_m.astype(jnp.int32) << 8)
                | q_l.astype(jnp.int32))
        q = jax.lax.bitcast_convert_type(bits, jnp.float32)
        qs = qs + q
        r = r - q
    zq_ref[...] = z + (qs - z)
    diff = qs - z
    sse_ref[...] = jnp.full((1, 1, 128), jnp.sum(diff * diff),
                            dtype=jnp.float32)


@jax.jit
def kernel(z_e, codebooks):
    n_tiles = N_TOKENS // TILE
    cb_bits = jax.lax.bitcast_convert_type(codebooks, jnp.int32)
    cb_hi = jax.lax.bitcast_convert_type(
        (cb_bits >> 16).astype(jnp.uint16), jnp.bfloat16)
    cb_m = ((cb_bits >> 8) & 0xFF).astype(jnp.bfloat16)
    cb_l = (cb_bits & 0xFF).astype(jnp.bfloat16)

    cb_spec = pl.BlockSpec((NUM_CODEBOOKS, CODEBOOK_SIZE, DIM),
                           lambda i: (0, 0, 0))
    zq, codes, sse = pl.pallas_call(
        _rvq_kernel,
        grid=(n_tiles,),
        in_specs=[
            pl.BlockSpec((TILE, DIM), lambda i: (i, 0)),
            cb_spec, cb_spec, cb_spec, cb_spec,
        ],
        out_specs=[
            pl.BlockSpec((TILE, DIM), lambda i: (i, 0)),
            pl.BlockSpec((TILE, NUM_CODEBOOKS), lambda i: (i, 0)),
            pl.BlockSpec((1, 1, 128), lambda i: (i, 0, 0)),
        ],
        out_shape=[
            jax.ShapeDtypeStruct((N_TOKENS, DIM), jnp.float32),
            jax.ShapeDtypeStruct((N_TOKENS, NUM_CODEBOOKS), jnp.int32),
            jax.ShapeDtypeStruct((n_tiles, 1, 128), jnp.float32),
        ],
        compiler_params=pltpu.CompilerParams(
            dimension_semantics=("arbitrary",)),
    )(z_e, codebooks, cb_hi, cb_m, cb_l)
    loss = (1.0 + COMMIT_COST) * jnp.sum(sse[:, 0, 0]) / (N_TOKENS * DIM)
    return (zq, codes, loss)
